# Initial kernel scaffold; baseline (speedup 1.0000x reference)
import jax
import jax.numpy as jnp
from jax.experimental import pallas as pl


def kernel(x, edge_index, edge_attr, W_enc, b_enc, prelu_a, edge_emb1, edge_emb2, W1, b1, W2, b2):
    raise NotImplementedError("write your pallas kernel here")


# speedup vs baseline: 2.2613x; 2.2613x over previous
import functools

import jax
import jax.numpy as jnp
from jax import lax
from jax.experimental import pallas as pl
from jax.experimental.pallas import tpu as pltpu
from jax.experimental.pallas import tpu_sc as plsc

NC = 2
NS = 16
NW = NC * NS

ROWB = 1000
K = 80


def _tc_pre_body(x_ref, w_ref, b_ref, a_ref, e1_ref, e2_ref, h_ref, t_ref):
    h = jnp.dot(x_ref[...], w_ref[...], preferred_element_type=jnp.float32)
    h = h + b_ref[...]
    a = a_ref[0, 0]
    h_ref[...] = jnp.where(h > 0, h, a * h)
    t_ref[...] = e1_ref[...][:, None, :] + e2_ref[...][None, :, :]


def _tc_post_body(p0_ref, p1_ref, w1_ref, b1_ref, w2_ref, b2_ref, o_ref):
    agg = p0_ref[...] + p1_ref[...]
    hid = jnp.dot(agg, w1_ref[...], preferred_element_type=jnp.float32) + b1_ref[...]
    hid = jnp.maximum(hid, 0.0)
    o_ref[...] = jnp.dot(hid, w2_ref[...], preferred_element_type=jnp.float32) + b2_ref[...]


@functools.cache
def _make_tc_pre(n, d, h):
    grid = (n // ROWB,)
    return pl.pallas_call(
        _tc_pre_body,
        grid=grid,
        in_specs=[
            pl.BlockSpec((ROWB, d), lambda i: (i, 0)),
            pl.BlockSpec((d, h), lambda i: (0, 0)),
            pl.BlockSpec((1, h), lambda i: (0, 0)),
            pl.BlockSpec((1, 1), lambda i: (0, 0), memory_space=pltpu.SMEM),
            pl.BlockSpec((8, h), lambda i: (0, 0)),
            pl.BlockSpec((8, h), lambda i: (0, 0)),
        ],
        out_specs=[
            pl.BlockSpec((ROWB, h), lambda i: (i, 0)),
            pl.BlockSpec((8, 8, h), lambda i: (0, 0, 0)),
        ],
        out_shape=[
            jax.ShapeDtypeStruct((n, h), jnp.float32),
            jax.ShapeDtypeStruct((8, 8, h), jnp.float32),
        ],
    )


@functools.cache
def _make_tc_post(n, h, h2, outd):
    grid = (n // ROWB,)
    return pl.pallas_call(
        _tc_post_body,
        grid=grid,
        in_specs=[
            pl.BlockSpec((ROWB, h), lambda i: (i, 0)),
            pl.BlockSpec((ROWB, h), lambda i: (i, 0)),
            pl.BlockSpec((h, h2), lambda i: (0, 0)),
            pl.BlockSpec((1, h2), lambda i: (0, 0)),
            pl.BlockSpec((h2, outd), lambda i: (0, 0)),
            pl.BlockSpec((1, outd), lambda i: (0, 0)),
        ],
        out_specs=pl.BlockSpec((ROWB, outd), lambda i: (i, 0)),
        out_shape=jax.ShapeDtypeStruct((n, outd), jnp.float32),
    )


@functools.cache
def _make_sc(n, e, h):
    epw = e // NW
    nchunk = epw // K
    rps = n // NS
    assert rps % 8 == 0 and n % NS == 0
    mesh = plsc.VectorSubcoreMesh(core_axis_name="c", subcore_axis_name="s")

    @functools.partial(
        pl.kernel,
        mesh=mesh,
        out_type=jax.ShapeDtypeStruct((NC, n, h), jnp.float32),
        scratch_types=[
            pltpu.VMEM((1, K), jnp.int32),
            pltpu.VMEM((1, K), jnp.int32),
            pltpu.VMEM((1, K), jnp.int32),
            pltpu.VMEM((1, K), jnp.int32),
            pltpu.VMEM((K, h), jnp.float32),
            pltpu.VMEM((K, h), jnp.float32),
            pltpu.VMEM_SHARED((n, h), jnp.float32),
            pltpu.SemaphoreType.DMA,
            pltpu.SemaphoreType.DMA,
        ],
    )
    def sck(h_hbm, src_hbm, dst_hbm, a0_hbm, a1_hbm, t_hbm, z_hbm, out_hbm,
            src_v, dst_v, a0_v, a1_v, hrows_v, trows_v, agg_sh, sem1, sem2):
        cid = lax.axis_index("c")
        sid = lax.axis_index("s")
        wid = sid * NC + cid
        base = wid * epw
        r0 = sid * rps

        pltpu.sync_copy(z_hbm.at[pl.ds(r0, rps)], agg_sh.at[pl.ds(r0, rps)])
        plsc.subcore_barrier()

        @pl.loop(0, nchunk)
        def _(ci):
            b = base + ci * K
            pltpu.sync_copy(src_hbm.at[pl.ds(b, K)], src_v.at[0])
            pltpu.sync_copy(dst_hbm.at[pl.ds(b, K)], dst_v.at[0])
            pltpu.sync_copy(a0_hbm.at[pl.ds(b, K)], a0_v.at[0])
            pltpu.sync_copy(a1_hbm.at[pl.ds(b, K)], a1_v.at[0])

            @pl.loop(0, K, step=16)
            def _(i):
                a0_v.at[0, pl.ds(i, 16)][...] = (
                    a0_v.at[0, pl.ds(i, 16)][...] * 8
                    + a1_v.at[0, pl.ds(i, 16)][...]
                )

            g1 = pltpu.async_copy(h_hbm.at[src_v.at[0]], hrows_v, sem1)
            g2 = pltpu.async_copy(t_hbm.at[a0_v.at[0]], trows_v, sem2)
            g1.wait()
            g2.wait()
            pltpu.sync_copy(hrows_v, agg_sh.at[dst_v.at[0]], add=True)
            pltpu.sync_copy(trows_v, agg_sh.at[dst_v.at[0]], add=True)

        plsc.subcore_barrier()
        pltpu.sync_copy(agg_sh.at[pl.ds(r0, rps)],
                        out_hbm.at[cid, pl.ds(r0, rps)])

    return sck


def kernel(x, edge_index, edge_attr, W_enc, b_enc, prelu_a,
           edge_emb1, edge_emb2, W1, b1, W2, b2):
    n, d = x.shape
    h = W_enc.shape[1]
    e = edge_index.shape[1]
    h2 = W1.shape[1]
    outd = W2.shape[1]

    e1p = jnp.zeros((8, h), jnp.float32).at[: edge_emb1.shape[0]].set(edge_emb1)
    e2p = jnp.zeros((8, h), jnp.float32).at[: edge_emb2.shape[0]].set(edge_emb2)

    hm, t3 = _make_tc_pre(n, d, h)(
        x, W_enc, b_enc.reshape(1, h), prelu_a.reshape(1, 1), e1p, e2p)
    tflat = t3.reshape(64, h)

    src = edge_index[0]
    dst = edge_index[1]
    a0 = edge_attr[:, 0]
    a1 = edge_attr[:, 1]

    npad = -(-n // (8 * NS)) * (8 * NS)
    zeros = jnp.zeros((npad, h), jnp.float32)

    partials = _make_sc(npad, e, h)(hm, src, dst, a0, a1, tflat, zeros)

    out = _make_tc_post(n, h, h2, outd)(
        partials[0, :n], partials[1, :n], W1, b1.reshape(1, h2), W2,
        b2.reshape(1, outd))
    return out


# speedup vs baseline: 2.2785x; 1.0076x over previous
import functools

import jax
import jax.numpy as jnp
from jax import lax
from jax.experimental import pallas as pl
from jax.experimental.pallas import tpu as pltpu
from jax.experimental.pallas import tpu_sc as plsc

NC = 2
NS = 16
NW = NC * NS

ROWB = 1000
K = 80


def _tc_pre_body(x_ref, w_ref, b_ref, a_ref, e1_ref, e2_ref, h_ref, t_ref):
    h = jnp.dot(x_ref[...], w_ref[...], preferred_element_type=jnp.float32)
    h = h + b_ref[...]
    a = a_ref[0, 0]
    h_ref[...] = jnp.where(h > 0, h, a * h)
    t_ref[...] = e1_ref[...][:, None, :] + e2_ref[...][None, :, :]


def _tc_post_body(p0_ref, p1_ref, w1_ref, b1_ref, w2_ref, b2_ref, o_ref):
    agg = p0_ref[...] + p1_ref[...]
    hid = jnp.dot(agg, w1_ref[...], preferred_element_type=jnp.float32) + b1_ref[...]
    hid = jnp.maximum(hid, 0.0)
    o_ref[...] = jnp.dot(hid, w2_ref[...], preferred_element_type=jnp.float32) + b2_ref[...]


@functools.cache
def _make_tc_pre(n, d, h):
    grid = (n // ROWB,)
    return pl.pallas_call(
        _tc_pre_body,
        grid=grid,
        in_specs=[
            pl.BlockSpec((ROWB, d), lambda i: (i, 0)),
            pl.BlockSpec((d, h), lambda i: (0, 0)),
            pl.BlockSpec((1, h), lambda i: (0, 0)),
            pl.BlockSpec((1, 1), lambda i: (0, 0), memory_space=pltpu.SMEM),
            pl.BlockSpec((8, h), lambda i: (0, 0)),
            pl.BlockSpec((8, h), lambda i: (0, 0)),
        ],
        out_specs=[
            pl.BlockSpec((ROWB, h), lambda i: (i, 0)),
            pl.BlockSpec((8, 8, h), lambda i: (0, 0, 0)),
        ],
        out_shape=[
            jax.ShapeDtypeStruct((n, h), jnp.float32),
            jax.ShapeDtypeStruct((8, 8, h), jnp.float32),
        ],
    )


@functools.cache
def _make_tc_post(n, h, h2, outd):
    grid = (n // ROWB,)
    return pl.pallas_call(
        _tc_post_body,
        grid=grid,
        in_specs=[
            pl.BlockSpec((ROWB, h), lambda i: (i, 0)),
            pl.BlockSpec((ROWB, h), lambda i: (i, 0)),
            pl.BlockSpec((h, h2), lambda i: (0, 0)),
            pl.BlockSpec((1, h2), lambda i: (0, 0)),
            pl.BlockSpec((h2, outd), lambda i: (0, 0)),
            pl.BlockSpec((1, outd), lambda i: (0, 0)),
        ],
        out_specs=pl.BlockSpec((ROWB, outd), lambda i: (i, 0)),
        out_shape=jax.ShapeDtypeStruct((n, outd), jnp.float32),
    )


@functools.cache
def _make_sc(n, e, h):
    epw = e // NW
    nchunk = epw // K
    rps = n // NS
    assert rps % 8 == 0 and n % NS == 0
    assert nchunk % 2 == 1 and nchunk >= 3
    mesh = plsc.VectorSubcoreMesh(core_axis_name="c", subcore_axis_name="s")

    @functools.partial(
        pl.kernel,
        mesh=mesh,
        out_type=jax.ShapeDtypeStruct((NC, n, h), jnp.float32),
        scratch_types=[
            pltpu.VMEM((4, K), jnp.int32),
            pltpu.VMEM((4, K), jnp.int32),
            pltpu.VMEM((K, h), jnp.float32),
            pltpu.VMEM((K, h), jnp.float32),
            pltpu.VMEM((K, h), jnp.float32),
            pltpu.VMEM((K, h), jnp.float32),
            pltpu.VMEM_SHARED((n, h), jnp.float32),
            pltpu.SemaphoreType.DMA,
            pltpu.SemaphoreType.DMA,
            pltpu.SemaphoreType.DMA,
            pltpu.SemaphoreType.DMA,
            pltpu.SemaphoreType.DMA,
            pltpu.SemaphoreType.DMA,
        ],
    )
    def sck(h_hbm, src_hbm, dst_hbm, a0_hbm, a1_hbm, t_hbm, z_hbm, out_hbm,
            idx0, idx1, hr0, hr1, tr0, tr1, agg_sh,
            si0, si1, sg0, sg1, ss0, ss1):
        cid = lax.axis_index("c")
        sid = lax.axis_index("s")
        wid = sid * NC + cid
        base = wid * epw
        r0 = sid * rps

        idx = (idx0, idx1)
        hr = (hr0, hr1)
        tr = (tr0, tr1)
        si = (si0, si1)
        sg = (sg0, sg1)
        ss = (ss0, ss1)

        pltpu.sync_copy(z_hbm.at[pl.ds(r0, rps)], agg_sh.at[pl.ds(r0, rps)])
        plsc.subcore_barrier()

        streams = (src_hbm, dst_hbm, a0_hbm, a1_hbm)

        def issue_idx(ci, b):
            o = base + ci * K
            for r, s_hbm in enumerate(streams):
                pltpu.async_copy(s_hbm.at[pl.ds(o, K)], idx[b].at[r], si[b])

        def wait_idx(ci, b):
            o = base + ci * K
            for r, s_hbm in enumerate(streams):
                pltpu.make_async_copy(
                    s_hbm.at[pl.ds(o, K)], idx[b].at[r], si[b]).wait()

        def compute_c(b):
            @pl.loop(0, K, step=16)
            def _(i):
                idx[b].at[2, pl.ds(i, 16)][...] = (
                    idx[b].at[2, pl.ds(i, 16)][...] * 8
                    + idx[b].at[3, pl.ds(i, 16)][...]
                )

        def issue_gathers(b):
            pltpu.async_copy(h_hbm.at[idx[b].at[0]], hr[b], sg[b])
            pltpu.async_copy(t_hbm.at[idx[b].at[2]], tr[b], sg[b])

        def wait_gathers(b):
            pltpu.make_async_copy(h_hbm.at[idx[b].at[0]], hr[b], sg[b]).wait()
            pltpu.make_async_copy(t_hbm.at[idx[b].at[2]], tr[b], sg[b]).wait()

        def issue_scatters(b):
            pltpu.async_copy(hr[b], agg_sh.at[idx[b].at[1]], ss[b], add=True)
            pltpu.async_copy(tr[b], agg_sh.at[idx[b].at[1]], ss[b], add=True)

        def wait_scatters(b):
            pltpu.make_async_copy(hr[b], agg_sh.at[idx[b].at[1]], ss[b]).wait()
            pltpu.make_async_copy(tr[b], agg_sh.at[idx[b].at[1]], ss[b]).wait()

        issue_idx(0, 0)
        issue_idx(1, 1)
        wait_idx(0, 0)
        compute_c(0)
        issue_gathers(0)

        def step(ci, cur, nxt):
            wait_idx(ci + 1, nxt)
            compute_c(nxt)
            issue_gathers(nxt)
            wait_gathers(cur)
            issue_scatters(cur)
            wait_scatters(cur)

            @pl.when(ci + 2 < nchunk)
            def _():
                issue_idx(ci + 2, cur)

        @pl.loop(0, nchunk - 1, step=2)
        def _(ci):
            step(ci, 0, 1)
            step(ci + 1, 1, 0)

        wait_gathers(0)
        issue_scatters(0)
        wait_scatters(0)

        plsc.subcore_barrier()
        pltpu.sync_copy(agg_sh.at[pl.ds(r0, rps)],
                        out_hbm.at[cid, pl.ds(r0, rps)])

    return sck


def kernel(x, edge_index, edge_attr, W_enc, b_enc, prelu_a,
           edge_emb1, edge_emb2, W1, b1, W2, b2):
    n, d = x.shape
    h = W_enc.shape[1]
    e = edge_index.shape[1]
    h2 = W1.shape[1]
    outd = W2.shape[1]

    e1p = jnp.zeros((8, h), jnp.float32).at[: edge_emb1.shape[0]].set(edge_emb1)
    e2p = jnp.zeros((8, h), jnp.float32).at[: edge_emb2.shape[0]].set(edge_emb2)

    hm, t3 = _make_tc_pre(n, d, h)(
        x, W_enc, b_enc.reshape(1, h), prelu_a.reshape(1, 1), e1p, e2p)
    tflat = t3.reshape(64, h)

    src = edge_index[0]
    dst = edge_index[1]
    a0 = edge_attr[:, 0]
    a1 = edge_attr[:, 1]

    npad = -(-n // (8 * NS)) * (8 * NS)
    zeros = jnp.zeros((npad, h), jnp.float32)

    partials = _make_sc(npad, e, h)(hm, src, dst, a0, a1, tflat, zeros)

    out = _make_tc_post(n, h, h2, outd)(
        partials[0, :n], partials[1, :n], W1, b1.reshape(1, h2), W2,
        b2.reshape(1, outd))
    return out


# speedup vs baseline: 7.6023x; 3.3366x over previous
import dataclasses
import functools

import jax
import jax.numpy as jnp
from jax import lax
from jax.experimental import pallas as pl
from jax.experimental.pallas import tpu as pltpu
from jax.experimental.pallas import tpu_sc as plsc

NC = 2
NS = 16
NW = NC * NS

ROWB = 1000
K = 80
CW = 16


def _tc_pre_body(x_ref, w_ref, b_ref, a_ref, h_ref):
    h = jnp.dot(x_ref[...], w_ref[...], preferred_element_type=jnp.float32)
    h = h + b_ref[...]
    a = a_ref[0, 0]
    h_ref[...] = jnp.where(h > 0, h, a * h)


def _tc_post_body(p0_ref, p1_ref, cs_ref,
                  e1_ref, e2_ref, w1_ref, b1_ref, w2_ref, b2_ref, o_ref):
    agg = p0_ref[...] + p1_ref[...]
    cnt = jnp.sum(cs_ref[...], axis=0)
    r = lax.broadcasted_iota(jnp.int32, (9, CW), 0)
    col = lax.broadcasted_iota(jnp.int32, (9, CW), 1)
    a1m = jnp.where(r // 3 == col, 1.0, 0.0)
    a2m = jnp.where(r % 3 == col, 1.0, 0.0)
    agg = agg + jnp.dot(jnp.dot(cnt, a1m, preferred_element_type=jnp.float32),
                        e1_ref[...], preferred_element_type=jnp.float32)
    agg = agg + jnp.dot(jnp.dot(cnt, a2m, preferred_element_type=jnp.float32),
                        e2_ref[...], preferred_element_type=jnp.float32)
    hid = jnp.dot(agg, w1_ref[...], preferred_element_type=jnp.float32) + b1_ref[...]
    hid = jnp.maximum(hid, 0.0)
    o_ref[...] = jnp.dot(hid, w2_ref[...], preferred_element_type=jnp.float32) + b2_ref[...]


@functools.cache
def _make_tc_pre(n, d, h):
    return pl.pallas_call(
        _tc_pre_body,
        grid=(n // ROWB,),
        in_specs=[
            pl.BlockSpec((ROWB, d), lambda i: (i, 0)),
            pl.BlockSpec((d, h), lambda i: (0, 0)),
            pl.BlockSpec((1, h), lambda i: (0, 0)),
            pl.BlockSpec((1, 1), lambda i: (0, 0), memory_space=pltpu.SMEM),
        ],
        out_specs=pl.BlockSpec((ROWB, h), lambda i: (i, 0)),
        out_shape=jax.ShapeDtypeStruct((n, h), jnp.float32),
    )


@functools.cache
def _make_tc_post(n, h, h2, outd):
    return pl.pallas_call(
        _tc_post_body,
        grid=(n // ROWB,),
        in_specs=[
            pl.BlockSpec((ROWB, h), lambda i: (i, 0)),
            pl.BlockSpec((ROWB, h), lambda i: (i, 0)),
            pl.BlockSpec((NW, ROWB, 9), lambda i: (0, i, 0)),
            pl.BlockSpec((CW, h), lambda i: (0, 0)),
            pl.BlockSpec((CW, h), lambda i: (0, 0)),
            pl.BlockSpec((h, h2), lambda i: (0, 0)),
            pl.BlockSpec((1, h2), lambda i: (0, 0)),
            pl.BlockSpec((h2, outd), lambda i: (0, 0)),
            pl.BlockSpec((1, outd), lambda i: (0, 0)),
        ],
        out_specs=pl.BlockSpec((ROWB, outd), lambda i: (i, 0)),
        out_shape=jax.ShapeDtypeStruct((n, outd), jnp.float32),
    )


@functools.cache
def _make_sc_rows(n, e, h):
    epw = e // NW
    nchunk = epw // K
    rps = n // NS
    assert rps % 8 == 0 and n % NS == 0
    assert nchunk % 2 == 1 and nchunk >= 3 and K % 16 == 0
    mesh = plsc.VectorSubcoreMesh(core_axis_name="c", subcore_axis_name="s")

    @functools.partial(
        pl.kernel,
        mesh=mesh,
        out_type=jax.ShapeDtypeStruct((NC, n, h), jnp.float32),
        scratch_types=[
            pltpu.VMEM((2, K), jnp.int32),
            pltpu.VMEM((2, K), jnp.int32),
            pltpu.VMEM((K, h), jnp.float32),
            pltpu.VMEM((K, h), jnp.float32),
            pltpu.VMEM_SHARED((n, h), jnp.float32),
            pltpu.SemaphoreType.DMA,
            pltpu.SemaphoreType.DMA,
            pltpu.SemaphoreType.DMA,
            pltpu.SemaphoreType.DMA,
            pltpu.SemaphoreType.DMA,
            pltpu.SemaphoreType.DMA,
        ],
    )
    def sck(h_hbm, src_hbm, dst_hbm, z_hbm, out_hbm,
            idx0, idx1, hr0, hr1, agg_sh, si0, si1, sg0, sg1, ss0, ss1):
        cid = lax.axis_index("c")
        sid = lax.axis_index("s")
        wid = sid * NC + cid
        base = wid * epw
        r0 = sid * rps

        idx = (idx0, idx1)
        hr = (hr0, hr1)
        si = (si0, si1)
        sg = (sg0, sg1)
        ss = (ss0, ss1)

        pltpu.sync_copy(z_hbm.at[pl.ds(r0, rps)], agg_sh.at[pl.ds(r0, rps)])
        plsc.subcore_barrier()

        streams = (src_hbm, dst_hbm)

        def issue_idx(ci, b):
            o = base + ci * K
            for r, s_hbm in enumerate(streams):
                pltpu.async_copy(s_hbm.at[pl.ds(o, K)], idx[b].at[r], si[b])

        def wait_idx(ci, b):
            o = base + ci * K
            for r, s_hbm in enumerate(streams):
                pltpu.make_async_copy(
                    s_hbm.at[pl.ds(o, K)], idx[b].at[r], si[b]).wait()

        def issue_gather(b):
            pltpu.async_copy(h_hbm.at[idx[b].at[0]], hr[b], sg[b])

        def wait_gather(b):
            pltpu.make_async_copy(h_hbm.at[idx[b].at[0]], hr[b], sg[b]).wait()

        def issue_scatter(b):
            pltpu.async_copy(hr[b], agg_sh.at[idx[b].at[1]], ss[b], add=True)

        def wait_scatter(b):
            pltpu.make_async_copy(hr[b], agg_sh.at[idx[b].at[1]], ss[b]).wait()

        issue_idx(0, 0)
        issue_idx(1, 1)
        wait_idx(0, 0)
        issue_gather(0)

        def step(ci, cur, nxt):
            wait_idx(ci + 1, nxt)
            issue_gather(nxt)
            wait_gather(cur)
            issue_scatter(cur)
            wait_scatter(cur)

            @pl.when(ci + 2 < nchunk)
            def _():
                issue_idx(ci + 2, cur)

        @pl.loop(0, nchunk - 1, step=2)
        def _(ci):
            step(ci, 0, 1)
            step(ci + 1, 1, 0)

        wait_gather(0)
        issue_scatter(0)
        wait_scatter(0)

        plsc.subcore_barrier()
        pltpu.sync_copy(agg_sh.at[pl.ds(r0, rps)],
                        out_hbm.at[cid, pl.ds(r0, rps)])

    return sck


@functools.cache
def _make_sc_counts(n, e):
    epw = e // NW
    nchunk = epw // K
    n9 = n * 9
    assert n9 % 8 == 0 and nchunk % 2 == 1 and K % 16 == 0
    mesh = plsc.VectorSubcoreMesh(core_axis_name="c", subcore_axis_name="s")
    cp = pltpu.CompilerParams()
    if "needs_layout_passes" in pltpu.CompilerParams.__dataclass_fields__:
        cp = dataclasses.replace(cp, needs_layout_passes=False)

    @functools.partial(
        pl.kernel,
        mesh=mesh,
        compiler_params=cp,
        out_type=jax.ShapeDtypeStruct((NW * n9,), jnp.float32),
        scratch_types=[
            pltpu.VMEM((3, K), jnp.int32),
            pltpu.VMEM((3, K), jnp.int32),
            pltpu.VMEM((n9,), jnp.float32),
            pltpu.SemaphoreType.DMA,
            pltpu.SemaphoreType.DMA,
        ],
    )
    def sck(dst_hbm, a0_hbm, a1_hbm, zc_hbm, out_hbm,
            idx0, idx1, cflat, si0, si1):
        cid = lax.axis_index("c")
        sid = lax.axis_index("s")
        wid = sid * NC + cid
        base = wid * epw

        idx = (idx0, idx1)
        si = (si0, si1)

        pltpu.sync_copy(zc_hbm, cflat)

        streams = (dst_hbm, a0_hbm, a1_hbm)

        def issue_idx(ci, b):
            o = base + ci * K
            for r, s_hbm in enumerate(streams):
                pltpu.async_copy(s_hbm.at[pl.ds(o, K)], idx[b].at[r], si[b])

        def wait_idx(ci, b):
            o = base + ci * K
            for r, s_hbm in enumerate(streams):
                pltpu.make_async_copy(
                    s_hbm.at[pl.ds(o, K)], idx[b].at[r], si[b]).wait()

        ones = jnp.ones((16,), jnp.float32)

        def accumulate(b):
            @pl.loop(0, K, step=16)
            def _(i):
                dv = idx[b].at[0, pl.ds(i, 16)][...]
                a0v = idx[b].at[1, pl.ds(i, 16)][...]
                a1v = idx[b].at[2, pl.ds(i, 16)][...]
                fidx = dv * 9 + a0v * 3 + a1v
                plsc.addupdate_scatter(cflat, [fidx], ones)

        issue_idx(0, 0)
        issue_idx(1, 1)

        def step(ci, cur, nxt):
            wait_idx(ci, cur)
            accumulate(cur)

            @pl.when(ci + 2 < nchunk)
            def _():
                issue_idx(ci + 2, cur)

        @pl.loop(0, nchunk - 1, step=2)
        def _(ci):
            step(ci, 0, 1)
            step(ci + 1, 1, 0)

        step(nchunk - 1, 0, 1)

        pltpu.sync_copy(cflat, out_hbm.at[pl.ds(wid * n9, n9)])

    return sck


def kernel(x, edge_index, edge_attr, W_enc, b_enc, prelu_a,
           edge_emb1, edge_emb2, W1, b1, W2, b2):
    n, d = x.shape
    h = W_enc.shape[1]
    e = edge_index.shape[1]
    h2 = W1.shape[1]
    outd = W2.shape[1]

    e1p = jnp.zeros((CW, h), jnp.float32).at[: edge_emb1.shape[0]].set(edge_emb1)
    e2p = jnp.zeros((CW, h), jnp.float32).at[: edge_emb2.shape[0]].set(edge_emb2)

    hm = _make_tc_pre(n, d, h)(
        x, W_enc, b_enc.reshape(1, h), prelu_a.reshape(1, 1))

    src = edge_index[0]
    dst = edge_index[1]
    a0 = edge_attr[:, 0]
    a1 = edge_attr[:, 1]

    npad = -(-n // (8 * NS)) * (8 * NS)
    zeros = jnp.zeros((npad, h), jnp.float32)
    zeros_c = jnp.zeros((n * 9,), jnp.float32)

    pr = _make_sc_rows(npad, e, h)(hm, src, dst, zeros)
    pc = _make_sc_counts(n, e)(dst, a0, a1, zeros_c)

    out = _make_tc_post(n, h, h2, outd)(
        pr[0, :n], pr[1, :n], pc.reshape(NW, n, 9),
        e1p, e2p, W1, b1.reshape(1, h2), W2, b2.reshape(1, outd))
    return out


# speedup vs baseline: 10.9097x; 1.4351x over previous
import dataclasses
import functools

import jax
import jax.numpy as jnp
from jax import lax
from jax.experimental import pallas as pl
from jax.experimental.pallas import tpu as pltpu
from jax.experimental.pallas import tpu_sc as plsc

NC = 2
NS = 16
NW = NC * NS

ROWB = 1000
K = 80
CW = 16


def _tc_pre_body(x_ref, w_ref, b_ref, a_ref, h_ref):
    h = jnp.dot(x_ref[...], w_ref[...], preferred_element_type=jnp.float32)
    h = h + b_ref[...]
    a = a_ref[0, 0]
    h_ref[...] = jnp.where(h > 0, h, a * h)


def _tc_post_body(p0_ref, p1_ref, cs_ref,
                  e1_ref, e2_ref, w1_ref, b1_ref, w2_ref, b2_ref, o_ref):
    agg = p0_ref[0] + p1_ref[0]
    cnt = jnp.sum(cs_ref[...], axis=0)
    r = lax.broadcasted_iota(jnp.int32, (9, CW), 0)
    col = lax.broadcasted_iota(jnp.int32, (9, CW), 1)
    a1m = jnp.where(r // 3 == col, 1.0, 0.0)
    a2m = jnp.where(r % 3 == col, 1.0, 0.0)
    agg = agg + jnp.dot(jnp.dot(cnt, a1m, preferred_element_type=jnp.float32),
                        e1_ref[...], preferred_element_type=jnp.float32)
    agg = agg + jnp.dot(jnp.dot(cnt, a2m, preferred_element_type=jnp.float32),
                        e2_ref[...], preferred_element_type=jnp.float32)
    hid = jnp.dot(agg, w1_ref[...], preferred_element_type=jnp.float32) + b1_ref[...]
    hid = jnp.maximum(hid, 0.0)
    o_ref[...] = jnp.dot(hid, w2_ref[...], preferred_element_type=jnp.float32) + b2_ref[...]


@functools.cache
def _make_tc_pre(n, d, h):
    return pl.pallas_call(
        _tc_pre_body,
        grid=(n // ROWB,),
        in_specs=[
            pl.BlockSpec((ROWB, d), lambda i: (i, 0)),
            pl.BlockSpec((d, h), lambda i: (0, 0)),
            pl.BlockSpec((1, h), lambda i: (0, 0)),
            pl.BlockSpec((1, 1), lambda i: (0, 0), memory_space=pltpu.SMEM),
        ],
        out_specs=pl.BlockSpec((ROWB, h), lambda i: (i, 0)),
        out_shape=jax.ShapeDtypeStruct((n, h), jnp.float32),
    )


@functools.cache
def _make_tc_post(n, npad, h, h2, outd):
    return pl.pallas_call(
        _tc_post_body,
        grid=(n // ROWB,),
        in_specs=[
            pl.BlockSpec((1, ROWB, h), lambda i: (0, i, 0)),
            pl.BlockSpec((1, ROWB, h), lambda i: (1, i, 0)),
            pl.BlockSpec((NW, ROWB, 9), lambda i: (0, i, 0)),
            pl.BlockSpec((CW, h), lambda i: (0, 0)),
            pl.BlockSpec((CW, h), lambda i: (0, 0)),
            pl.BlockSpec((h, h2), lambda i: (0, 0)),
            pl.BlockSpec((1, h2), lambda i: (0, 0)),
            pl.BlockSpec((h2, outd), lambda i: (0, 0)),
            pl.BlockSpec((1, outd), lambda i: (0, 0)),
        ],
        out_specs=pl.BlockSpec((ROWB, outd), lambda i: (i, 0)),
        out_shape=jax.ShapeDtypeStruct((n, outd), jnp.float32),
    )


@functools.cache
def _make_sc_rows(n, e, h):
    epw = e // NW
    nchunk = epw // K
    rps = n // NS
    assert rps % 8 == 0 and n % NS == 0
    assert nchunk % 2 == 1 and nchunk >= 3 and K % 16 == 0
    mesh = plsc.VectorSubcoreMesh(core_axis_name="c", subcore_axis_name="s")

    @functools.partial(
        pl.kernel,
        mesh=mesh,
        out_type=jax.ShapeDtypeStruct((NC, n, h), jnp.float32),
        scratch_types=[
            pltpu.VMEM((2, K), jnp.int32),
            pltpu.VMEM((2, K), jnp.int32),
            pltpu.VMEM((K, h), jnp.float32),
            pltpu.VMEM((K, h), jnp.float32),
            pltpu.VMEM_SHARED((n, h), jnp.float32),
            pltpu.SemaphoreType.DMA,
            pltpu.SemaphoreType.DMA,
            pltpu.SemaphoreType.DMA,
            pltpu.SemaphoreType.DMA,
            pltpu.SemaphoreType.DMA,
            pltpu.SemaphoreType.DMA,
        ],
    )
    def sck(h_hbm, src_hbm, dst_hbm, z_hbm, out_hbm,
            idx0, idx1, hr0, hr1, agg_sh, si0, si1, sg0, sg1, ss0, ss1):
        cid = lax.axis_index("c")
        sid = lax.axis_index("s")
        wid = sid * NC + cid
        base = wid * epw
        r0 = sid * rps

        idx = (idx0, idx1)
        hr = (hr0, hr1)
        si = (si0, si1)
        sg = (sg0, sg1)
        ss = (ss0, ss1)

        pltpu.sync_copy(z_hbm.at[pl.ds(r0, rps)], agg_sh.at[pl.ds(r0, rps)])
        plsc.subcore_barrier()

        streams = (src_hbm, dst_hbm)

        def issue_idx(ci, b):
            o = base + ci * K
            for r, s_hbm in enumerate(streams):
                pltpu.async_copy(s_hbm.at[pl.ds(o, K)], idx[b].at[r], si[b])

        def wait_idx(ci, b):
            o = base + ci * K
            for r, s_hbm in enumerate(streams):
                pltpu.make_async_copy(
                    s_hbm.at[pl.ds(o, K)], idx[b].at[r], si[b]).wait()

        def issue_gather(b):
            pltpu.async_copy(h_hbm.at[idx[b].at[0]], hr[b], sg[b])

        def wait_gather(b):
            pltpu.make_async_copy(h_hbm.at[idx[b].at[0]], hr[b], sg[b]).wait()

        def issue_scatter(b):
            pltpu.async_copy(hr[b], agg_sh.at[idx[b].at[1]], ss[b], add=True)

        def wait_scatter(b):
            pltpu.make_async_copy(hr[b], agg_sh.at[idx[b].at[1]], ss[b]).wait()

        issue_idx(0, 0)
        issue_idx(1, 1)
        wait_idx(0, 0)
        issue_gather(0)

        def step(ci, cur, nxt):
            wait_idx(ci + 1, nxt)
            issue_gather(nxt)
            wait_gather(cur)
            issue_scatter(cur)
            wait_scatter(cur)

            @pl.when(ci + 2 < nchunk)
            def _():
                issue_idx(ci + 2, cur)

        @pl.loop(0, nchunk - 1, step=2)
        def _(ci):
            step(ci, 0, 1)
            step(ci + 1, 1, 0)

        wait_gather(0)
        issue_scatter(0)
        wait_scatter(0)

        plsc.subcore_barrier()
        pltpu.sync_copy(agg_sh.at[pl.ds(r0, rps)],
                        out_hbm.at[cid, pl.ds(r0, rps)])

    return sck


@functools.cache
def _make_sc_counts(n, e):
    epw = e // NW
    nchunk = epw // K
    n9 = n * 9
    assert n9 % 8 == 0 and nchunk % 2 == 1 and K % 16 == 0
    mesh = plsc.VectorSubcoreMesh(core_axis_name="c", subcore_axis_name="s")
    cp = pltpu.CompilerParams()
    if "needs_layout_passes" in pltpu.CompilerParams.__dataclass_fields__:
        cp = dataclasses.replace(cp, needs_layout_passes=False)

    @functools.partial(
        pl.kernel,
        mesh=mesh,
        compiler_params=cp,
        out_type=jax.ShapeDtypeStruct((NW * n9,), jnp.float32),
        scratch_types=[
            pltpu.VMEM((3, K), jnp.int32),
            pltpu.VMEM((3, K), jnp.int32),
            pltpu.VMEM((n9,), jnp.float32),
            pltpu.SemaphoreType.DMA,
            pltpu.SemaphoreType.DMA,
        ],
    )
    def sck(dst_hbm, a0_hbm, a1_hbm, zc_hbm, out_hbm,
            idx0, idx1, cflat, si0, si1):
        cid = lax.axis_index("c")
        sid = lax.axis_index("s")
        wid = sid * NC + cid
        base = wid * epw

        idx = (idx0, idx1)
        si = (si0, si1)

        pltpu.sync_copy(zc_hbm, cflat)

        streams = (dst_hbm, a0_hbm, a1_hbm)

        def issue_idx(ci, b):
            o = base + ci * K
            for r, s_hbm in enumerate(streams):
                pltpu.async_copy(s_hbm.at[pl.ds(o, K)], idx[b].at[r], si[b])

        def wait_idx(ci, b):
            o = base + ci * K
            for r, s_hbm in enumerate(streams):
                pltpu.make_async_copy(
                    s_hbm.at[pl.ds(o, K)], idx[b].at[r], si[b]).wait()

        ones = jnp.ones((16,), jnp.float32)

        def accumulate(b):
            @pl.loop(0, K, step=16)
            def _(i):
                dv = idx[b].at[0, pl.ds(i, 16)][...]
                a0v = idx[b].at[1, pl.ds(i, 16)][...]
                a1v = idx[b].at[2, pl.ds(i, 16)][...]
                fidx = dv * 9 + a0v * 3 + a1v
                plsc.addupdate_scatter(cflat, [fidx], ones)

        issue_idx(0, 0)
        issue_idx(1, 1)

        def step(ci, cur, nxt):
            wait_idx(ci, cur)
            accumulate(cur)

            @pl.when(ci + 2 < nchunk)
            def _():
                issue_idx(ci + 2, cur)

        @pl.loop(0, nchunk - 1, step=2)
        def _(ci):
            step(ci, 0, 1)
            step(ci + 1, 1, 0)

        step(nchunk - 1, 0, 1)

        pltpu.sync_copy(cflat, out_hbm.at[pl.ds(wid * n9, n9)])

    return sck


def kernel(x, edge_index, edge_attr, W_enc, b_enc, prelu_a,
           edge_emb1, edge_emb2, W1, b1, W2, b2):
    n, d = x.shape
    h = W_enc.shape[1]
    e = edge_index.shape[1]
    h2 = W1.shape[1]
    outd = W2.shape[1]

    e1p = jnp.zeros((CW, h), jnp.float32).at[: edge_emb1.shape[0]].set(edge_emb1)
    e2p = jnp.zeros((CW, h), jnp.float32).at[: edge_emb2.shape[0]].set(edge_emb2)

    src = edge_index[0]
    dst = edge_index[1]
    a0 = edge_attr[:, 0]
    a1 = edge_attr[:, 1]

    npad = -(-n // (8 * NS)) * (8 * NS)
    zeros = jnp.zeros((npad, h), jnp.float32)
    zeros_c = jnp.zeros((n * 9,), jnp.float32)

    pc = _make_sc_counts(n, e)(dst, a0, a1, zeros_c)

    hm = _make_tc_pre(n, d, h)(
        x, W_enc, b_enc.reshape(1, h), prelu_a.reshape(1, 1))

    pr = _make_sc_rows(npad, e, h)(hm, src, dst, zeros)

    out = _make_tc_post(n, npad, h, h2, outd)(
        pr, pr, pc.reshape(NW, n, 9),
        e1p, e2p, W1, b1.reshape(1, h2), W2, b2.reshape(1, outd))
    return out
